# baseline (speedup 1.0000x reference)
import functools

import jax
import jax.numpy as jnp
from jax import lax
from jax.experimental import pallas as pl
from jax.experimental.pallas import tpu as pltpu
from jax.experimental.pallas import tpu_sc as plsc

N = 320000
D = 128
G = 1024
OUT = 8

BLK = 6400
NBLK = N // BLK
N_PAD = N + BLK
TILES = N_PAD // 128

NW = 32
SEG_PER = G // NW
CHT = 4


def _mlp_body(x_ref, woutT_ref, wpack_ref, boutc_ref, bias_ref, out_ref):
    x = x_ref[...]
    wp = wpack_ref[...]
    wtransT = wp[0:8, :]
    wcombT = jnp.dot(wtransT, woutT_ref[...],
                     preferred_element_type=jnp.float32)
    wcatT = jnp.concatenate([wcombT, wp[8:16, :]], axis=0)
    yT = lax.dot_general(wcatT, x, (((1,), (1,)), ((), ())),
                         preferred_element_type=jnp.float32)
    b = bias_ref[...]
    bcT = jnp.dot(wtransT, boutc_ref[...],
                  preferred_element_type=jnp.float32) + b[:, 0:1]
    bg = b[0:1, 1:2]
    gate = jax.nn.sigmoid(yT[8:9, :] + bg)
    gatedT = gate * (yT[0:8, :] + bcT)
    for t in range(BLK // 128):
        out_ref[t] = gatedT[:, 128 * t:128 * (t + 1)]


def _gated_tiles(x, w_outT, wpack, b_out_col, bias_pack):
    return pl.pallas_call(
        _mlp_body,
        grid=(NBLK + 1,),
        in_specs=[
            pl.BlockSpec((BLK, D), lambda i: (jnp.minimum(i, NBLK - 1), 0)),
            pl.BlockSpec((D, D), lambda i: (0, 0)),
            pl.BlockSpec((16, D), lambda i: (0, 0)),
            pl.BlockSpec((D, 1), lambda i: (0, 0)),
            pl.BlockSpec((8, D), lambda i: (0, 0)),
        ],
        out_specs=pl.BlockSpec((BLK // 128, 8, 128), lambda i: (i, 0, 0)),
        out_shape=jax.ShapeDtypeStruct((TILES, 8, 128), jnp.float32),
    )(x, w_outT, wpack, b_out_col, bias_pack)


@functools.cache
def _make_segmean():
    return functools.partial(
        pl.kernel,
        out_type=jax.ShapeDtypeStruct((G * 16,), jnp.float32),
        mesh=plsc.VectorSubcoreMesh(core_axis_name="c", subcore_axis_name="s"),
        scratch_types=[
            pltpu.VMEM((48,), jnp.int32),
            pltpu.VMEM((CHT, 8, 128), jnp.float32),
            pltpu.VMEM((CHT, 8, 128), jnp.float32),
            pltpu.VMEM((CHT, 8, 128), jnp.float32),
            pltpu.SemaphoreType.DMA,
            pltpu.SemaphoreType.DMA,
            pltpu.VMEM((32,), jnp.float32),
            pltpu.VMEM((SEG_PER * 16,), jnp.float32),
        ],
    )(_segmean_body)


def _segmean_body(gated_hbm, splits_hbm, out_hbm, splits_v,
                  buf0, buf1, bufc, sem0, sem1, tmp, res):
    cid = lax.axis_index("c")
    sid = lax.axis_index("s")
    t = sid * 2 + cid
    lane = lax.iota(jnp.int32, 16)
    zero16 = jnp.zeros((16,), jnp.float32)
    pltpu.sync_copy(splits_hbm.at[pl.ds(t * SEG_PER, 40)],
                    splits_v.at[pl.ds(0, 40)])

    def acc_chunk(buf, tbase, ntiles, s, e, accs0):
        def tile_body(tt, accs2):
            colbase = (tbase + tt) * 128
            rel_s = jnp.clip(s - colbase, 0, 128)
            rel_e = jnp.clip(e - colbase, 0, 128)
            jlo = lax.shift_right_logical(rel_s + 15, 4)
            jhi = lax.shift_right_logical(rel_e, 4)

            def full_j(j, a8):
                return tuple(
                    a8[r] + buf[tt, r, pl.ds(16 * j, 16)] for r in range(8))

            def masked_j(j, a8):
                colv = jnp.full((16,), colbase + 16 * j, jnp.int32) + lane
                m = (colv >= s) & (colv < e)
                return tuple(
                    a8[r] + jnp.where(m, buf[tt, r, pl.ds(16 * j, 16)],
                                      zero16)
                    for r in range(8))

            accs2 = lax.fori_loop(jlo, jhi, full_j, accs2)
            accs2 = lax.fori_loop(lax.shift_right_logical(rel_s, 4), jlo,
                                  masked_j, accs2)
            accs2 = lax.fori_loop(jnp.maximum(jhi, jlo),
                                  lax.shift_right_logical(rel_e + 15, 4),
                                  masked_j, accs2)
            return accs2

        return lax.fori_loop(0, ntiles, tile_body, accs0)

    sv0 = splits_v[pl.ds(0, 16)]
    pltpu.async_copy(
        gated_hbm.at[pl.ds(lax.shift_right_logical(sv0[0], 7), CHT)],
        buf0, sem0)

    def run_seg(g, buf_a, sem_a, buf_b, sem_b, prefetch):
        sv = splits_v[pl.ds(g, 16)]
        s = sv[0]
        e = sv[1]
        nrows = e - s
        t0 = lax.shift_right_logical(s, 7)
        t1 = lax.shift_right_logical(e + 127, 7)
        nch = lax.shift_right_logical(t1 - t0 + (CHT - 1), 2)

        @pl.when(prefetch)
        def _prefetch():
            pltpu.async_copy(
                gated_hbm.at[pl.ds(lax.shift_right_logical(e, 7), CHT)],
                buf_b, sem_b)

        pltpu.make_async_copy(gated_hbm.at[pl.ds(0, CHT)],
                              buf_a, sem_a).wait()
        accs = acc_chunk(buf_a, t0, jnp.minimum(t1 - t0, CHT), s, e,
                         (zero16,) * 8)

        def cb(ci, accs2):
            tb = t0 + ci * CHT
            pltpu.sync_copy(gated_hbm.at[pl.ds(tb, CHT)], bufc)
            return acc_chunk(bufc, tb, jnp.minimum(t1 - tb, CHT), s, e,
                             accs2)

        accs = lax.fori_loop(1, nch, cb, accs)
        denom = jnp.maximum(nrows, 1).astype(jnp.float32)
        resv = zero16
        for r in range(8):
            a = accs[r]
            for shift in (8, 4, 2, 1):
                tmp[pl.ds(0, 16)] = a
                a = a + tmp[pl.ds(shift, 16)]
            resv = jnp.where(lane == r, a[0], resv)
        res[pl.ds(g * 16, 16)] = resv / denom

    def pair_body(p, carry):
        g0 = 2 * p
        run_seg(g0, buf0, sem0, buf1, sem1, g0 < SEG_PER - 1)
        run_seg(g0 + 1, buf1, sem1, buf0, sem0, g0 + 1 < SEG_PER - 1)
        return carry

    lax.fori_loop(0, SEG_PER // 2, pair_body, 0)
    pltpu.sync_copy(res.at[pl.ds(0, SEG_PER * 16)],
                    out_hbm.at[pl.ds(t * SEG_PER * 16, SEG_PER * 16)])


def kernel(flat_features, row_splits, W_out, b_out, W_gate, b_gate, W_trans, b_trans):
    w_outT = W_out.astype(jnp.float32).T
    wpack = jnp.zeros((16, D), jnp.float32)
    wpack = wpack.at[0:8, :].set(W_trans.astype(jnp.float32).T)
    wpack = wpack.at[8, :].set(W_gate.astype(jnp.float32)[:, 0])
    b_out_col = b_out.astype(jnp.float32).reshape(D, 1)
    bias_pack = jnp.zeros((8, D), jnp.float32)
    bias_pack = bias_pack.at[:, 0].set(b_trans.astype(jnp.float32))
    bias_pack = bias_pack.at[0, 1].set(b_gate.astype(jnp.float32)[0])

    gated = _gated_tiles(flat_features, w_outT, wpack, b_out_col, bias_pack)

    splits_p = jnp.concatenate(
        [row_splits.astype(jnp.int32), jnp.full((7,), N, jnp.int32)])

    out_flat = _make_segmean()(gated, splits_p)
    return out_flat.reshape(G, 16)[:, :OUT]


# speedup vs baseline: 1.0544x; 1.0544x over previous
import functools

import jax
import jax.numpy as jnp
from jax import lax
from jax.experimental import pallas as pl
from jax.experimental.pallas import tpu as pltpu
from jax.experimental.pallas import tpu_sc as plsc

N = 320000
D = 128
G = 1024
OUT = 8

BLK = 6400
NBLK = N // BLK
N_PAD = N + BLK
TILES = N_PAD // 128

NW = 32
SEG_PER = G // NW
CHT = 4


def _mlp_body(x_ref, woutT_ref, wpack_ref, boutc_ref, bias_ref, out_ref):
    x = x_ref[...]
    wp = wpack_ref[...]
    wtransT = wp[0:8, :]
    wcombT = jnp.dot(wtransT, woutT_ref[...],
                     preferred_element_type=jnp.float32)
    wcatT = jnp.concatenate([wcombT, wp[8:16, :]], axis=0)
    yT = lax.dot_general(wcatT, x, (((1,), (1,)), ((), ())),
                         preferred_element_type=jnp.float32)
    b = bias_ref[...]
    bcT = jnp.dot(wtransT, boutc_ref[...],
                  preferred_element_type=jnp.float32) + b[:, 0:1]
    bg = b[0:1, 1:2]
    gate = jax.nn.sigmoid(yT[8:9, :] + bg)
    gatedT = gate * (yT[0:8, :] + bcT)
    for t in range(BLK // 128):
        out_ref[t] = gatedT[:, 128 * t:128 * (t + 1)]


def _gated_tiles(x, w_outT, wpack, b_out_col, bias_pack):
    return pl.pallas_call(
        _mlp_body,
        grid=(NBLK + 1,),
        in_specs=[
            pl.BlockSpec((BLK, D), lambda i: (jnp.minimum(i, NBLK - 1), 0)),
            pl.BlockSpec((D, D), lambda i: (0, 0)),
            pl.BlockSpec((16, D), lambda i: (0, 0)),
            pl.BlockSpec((D, 1), lambda i: (0, 0)),
            pl.BlockSpec((8, D), lambda i: (0, 0)),
        ],
        out_specs=pl.BlockSpec((BLK // 128, 8, 128), lambda i: (i, 0, 0)),
        out_shape=jax.ShapeDtypeStruct((TILES, 8, 128), jnp.float32),
    )(x, w_outT, wpack, b_out_col, bias_pack)


@functools.cache
def _make_segmean():
    return functools.partial(
        pl.kernel,
        out_type=jax.ShapeDtypeStruct((G * 16,), jnp.float32),
        mesh=plsc.VectorSubcoreMesh(core_axis_name="c", subcore_axis_name="s"),
        scratch_types=[
            pltpu.VMEM((48,), jnp.int32),
            pltpu.VMEM((CHT, 8, 128), jnp.float32),
            pltpu.VMEM((CHT, 8, 128), jnp.float32),
            pltpu.VMEM((CHT, 8, 128), jnp.float32),
            pltpu.VMEM((CHT, 8, 128), jnp.float32),
            pltpu.VMEM((CHT, 8, 128), jnp.float32),
            pltpu.SemaphoreType.DMA,
            pltpu.SemaphoreType.DMA,
            pltpu.SemaphoreType.DMA,
            pltpu.SemaphoreType.DMA,
            pltpu.VMEM((144,), jnp.float32),
            pltpu.VMEM((SEG_PER * 16,), jnp.float32),
        ],
    )(_segmean_body)


def _segmean_body(gated_hbm, splits_hbm, out_hbm, splits_v,
                  buf0, buf1, buf2, buf3, bufc,
                  sem0, sem1, sem2, sem3, tmp, res):
    cid = lax.axis_index("c")
    sid = lax.axis_index("s")
    t = sid * 2 + cid
    lane = lax.iota(jnp.int32, 16)
    zero16 = jnp.zeros((16,), jnp.float32)
    pltpu.sync_copy(splits_hbm.at[pl.ds(t * SEG_PER, 40)],
                    splits_v.at[pl.ds(0, 40)])

    def acc_chunk(buf, tbase, ntiles, s, e, accs0):
        def tile_body(tt, accs2):
            colbase = (tbase + tt) * 128
            cols = [jnp.full((16,), colbase + 16 * j, jnp.int32) + lane
                    for j in range(8)]
            masks = [(c >= s) & (c < e) for c in cols]
            out = []
            for r in range(8):
                a = accs2[r]
                for j in range(8):
                    v = buf[tt, r, pl.ds(16 * j, 16)]
                    a = a + jnp.where(masks[j], v, zero16)
                out.append(a)
            return tuple(out)

        return lax.fori_loop(0, ntiles, tile_body, accs0)

    sv0 = splits_v[pl.ds(0, 16)]
    pltpu.async_copy(
        gated_hbm.at[pl.ds(lax.shift_right_logical(sv0[0], 7), CHT)],
        buf0, sem0)
    pltpu.async_copy(
        gated_hbm.at[pl.ds(lax.shift_right_logical(sv0[1], 7), CHT)],
        buf1, sem1)

    def run_seg(g, buf_a, sem_a, buf_b, sem_b, prefetch):
        sv = splits_v[pl.ds(g, 16)]
        s = sv[0]
        e = sv[1]
        nrows = e - s
        t0 = lax.shift_right_logical(s, 7)
        t1 = lax.shift_right_logical(e + 127, 7)
        nch = lax.shift_right_logical(t1 - t0 + (CHT - 1), 2)

        @pl.when(prefetch)
        def _prefetch():
            pltpu.async_copy(
                gated_hbm.at[pl.ds(lax.shift_right_logical(sv[2], 7), CHT)],
                buf_b, sem_b)

        pltpu.make_async_copy(gated_hbm.at[pl.ds(0, CHT)],
                              buf_a, sem_a).wait()
        accs = acc_chunk(buf_a, t0, jnp.minimum(t1 - t0, CHT), s, e,
                         (zero16,) * 8)

        def cb(ci, accs2):
            tb = t0 + ci * CHT
            pltpu.sync_copy(gated_hbm.at[pl.ds(tb, CHT)], bufc)
            return acc_chunk(bufc, tb, jnp.minimum(t1 - tb, CHT), s, e,
                             accs2)

        accs = lax.fori_loop(1, nch, cb, accs)
        denom = jnp.maximum(nrows, 1).astype(jnp.float32)
        a8 = list(accs)
        for shift in (8, 4, 2, 1):
            for r in range(8):
                tmp[pl.ds(r * 16, 16)] = a8[r]
            for r in range(8):
                a8[r] = a8[r] + tmp[pl.ds(r * 16 + shift, 16)]
        resv = zero16
        for r in range(8):
            resv = jnp.where(lane == r, a8[r][0], resv)
        res[pl.ds(g * 16, 16)] = resv / denom

    def quad_body(p, carry):
        g0 = 4 * p
        run_seg(g0, buf0, sem0, buf2, sem2, g0 + 2 < SEG_PER)
        run_seg(g0 + 1, buf1, sem1, buf3, sem3, g0 + 3 < SEG_PER)
        run_seg(g0 + 2, buf2, sem2, buf0, sem0, g0 + 4 < SEG_PER)
        run_seg(g0 + 3, buf3, sem3, buf1, sem1, g0 + 5 < SEG_PER)
        return carry

    lax.fori_loop(0, SEG_PER // 4, quad_body, 0)
    pltpu.sync_copy(res.at[pl.ds(0, SEG_PER * 16)],
                    out_hbm.at[pl.ds(t * SEG_PER * 16, SEG_PER * 16)])


def kernel(flat_features, row_splits, W_out, b_out, W_gate, b_gate, W_trans, b_trans):
    w_outT = W_out.astype(jnp.float32).T
    wpack = jnp.zeros((16, D), jnp.float32)
    wpack = wpack.at[0:8, :].set(W_trans.astype(jnp.float32).T)
    wpack = wpack.at[8, :].set(W_gate.astype(jnp.float32)[:, 0])
    b_out_col = b_out.astype(jnp.float32).reshape(D, 1)
    bias_pack = jnp.zeros((8, D), jnp.float32)
    bias_pack = bias_pack.at[:, 0].set(b_trans.astype(jnp.float32))
    bias_pack = bias_pack.at[0, 1].set(b_gate.astype(jnp.float32)[0])

    gated = _gated_tiles(flat_features, w_outT, wpack, b_out_col, bias_pack)

    splits_p = jnp.concatenate(
        [row_splits.astype(jnp.int32), jnp.full((7,), N, jnp.int32)])

    out_flat = _make_segmean()(gated, splits_p)
    return out_flat.reshape(G, 16)[:, :OUT]


# speedup vs baseline: 1.1948x; 1.1332x over previous
import functools

import jax
import jax.numpy as jnp
from jax import lax
from jax.experimental import pallas as pl
from jax.experimental.pallas import tpu as pltpu
from jax.experimental.pallas import tpu_sc as plsc

N = 320000
D = 128
G = 1024
OUT = 8

BLK = 12800
NBLK = N // BLK
N_PAD = N + BLK
TILES = N_PAD // 128

NW = 32
SEG_PER = G // NW
CHT = 4


def _mlp_body(x_ref, woutT_ref, wpack_ref, boutc_ref, bias_ref, out_ref):
    x = x_ref[...]
    wp = wpack_ref[...]
    wtransT = wp[0:8, :]
    wcombT = jnp.dot(wtransT, woutT_ref[...],
                     preferred_element_type=jnp.float32)
    wcatT = jnp.concatenate([wcombT, wp[8:16, :]], axis=0)
    yT = lax.dot_general(wcatT, x, (((1,), (1,)), ((), ())),
                         preferred_element_type=jnp.float32)
    b = bias_ref[...]
    bcT = jnp.dot(wtransT, boutc_ref[...],
                  preferred_element_type=jnp.float32) + b[:, 0:1]
    bg = b[0:1, 1:2]
    gate = jax.nn.sigmoid(yT[8:9, :] + bg)
    gatedT = gate * (yT[0:8, :] + bcT)
    for t in range(BLK // 128):
        out_ref[t] = gatedT[:, 128 * t:128 * (t + 1)]


def _gated_tiles(x, w_outT, wpack, b_out_col, bias_pack):
    return pl.pallas_call(
        _mlp_body,
        grid=(NBLK + 1,),
        in_specs=[
            pl.BlockSpec((BLK, D), lambda i: (jnp.minimum(i, NBLK - 1), 0)),
            pl.BlockSpec((D, D), lambda i: (0, 0)),
            pl.BlockSpec((16, D), lambda i: (0, 0)),
            pl.BlockSpec((D, 1), lambda i: (0, 0)),
            pl.BlockSpec((8, D), lambda i: (0, 0)),
        ],
        out_specs=pl.BlockSpec((BLK // 128, 8, 128), lambda i: (i, 0, 0)),
        out_shape=jax.ShapeDtypeStruct((TILES, 8, 128), jnp.float32),
    )(x, w_outT, wpack, b_out_col, bias_pack)


@functools.cache
def _make_segmean():
    return functools.partial(
        pl.kernel,
        out_type=jax.ShapeDtypeStruct((G * 16,), jnp.float32),
        mesh=plsc.VectorSubcoreMesh(core_axis_name="c", subcore_axis_name="s"),
        scratch_types=[
            pltpu.VMEM((48,), jnp.int32),
            pltpu.VMEM((CHT, 8, 128), jnp.float32),
            pltpu.VMEM((CHT, 8, 128), jnp.float32),
            pltpu.VMEM((CHT, 8, 128), jnp.float32),
            pltpu.VMEM((CHT, 8, 128), jnp.float32),
            pltpu.VMEM((CHT, 8, 128), jnp.float32),
            pltpu.SemaphoreType.DMA,
            pltpu.SemaphoreType.DMA,
            pltpu.SemaphoreType.DMA,
            pltpu.SemaphoreType.DMA,
            pltpu.VMEM((144,), jnp.float32),
            pltpu.VMEM((SEG_PER * 16,), jnp.float32),
        ],
    )(_segmean_body)


def _segmean_body(gated_hbm, splits_hbm, out_hbm, splits_v,
                  buf0, buf1, buf2, buf3, bufc,
                  sem0, sem1, sem2, sem3, tmp, res):
    cid = lax.axis_index("c")
    sid = lax.axis_index("s")
    t = sid * 2 + cid
    lane = lax.iota(jnp.int32, 16)
    zero16 = jnp.zeros((16,), jnp.float32)
    pltpu.sync_copy(splits_hbm.at[pl.ds(t * SEG_PER, 40)],
                    splits_v.at[pl.ds(0, 40)])

    def acc_chunk(buf, tbase, ntiles, s, e, accs0):
        def tile_body(tt, accs2):
            colbase = (tbase + tt) * 128
            cols = [jnp.full((16,), colbase + 16 * j, jnp.int32) + lane
                    for j in range(8)]
            masks = [(c >= s) & (c < e) for c in cols]
            out = []
            for r in range(8):
                a = accs2[r]
                for j in range(8):
                    v = buf[tt, r, pl.ds(16 * j, 16)]
                    a = a + jnp.where(masks[j], v, zero16)
                out.append(a)
            return tuple(out)

        return lax.fori_loop(0, ntiles, tile_body, accs0)

    sv0 = splits_v[pl.ds(0, 16)]
    pltpu.async_copy(
        gated_hbm.at[pl.ds(lax.shift_right_logical(sv0[0], 7), CHT)],
        buf0, sem0)
    pltpu.async_copy(
        gated_hbm.at[pl.ds(lax.shift_right_logical(sv0[1], 7), CHT)],
        buf1, sem1)

    def run_seg(g, buf_a, sem_a, buf_b, sem_b, prefetch):
        sv = splits_v[pl.ds(g, 16)]
        s = sv[0]
        e = sv[1]
        nrows = e - s
        t0 = lax.shift_right_logical(s, 7)
        t1 = lax.shift_right_logical(e + 127, 7)
        nch = lax.shift_right_logical(t1 - t0 + (CHT - 1), 2)

        @pl.when(prefetch)
        def _prefetch():
            pltpu.async_copy(
                gated_hbm.at[pl.ds(lax.shift_right_logical(sv[2], 7), CHT)],
                buf_b, sem_b)

        pltpu.make_async_copy(gated_hbm.at[pl.ds(0, CHT)],
                              buf_a, sem_a).wait()
        accs = acc_chunk(buf_a, t0, jnp.minimum(t1 - t0, CHT), s, e,
                         (zero16,) * 8)

        def cb(ci, accs2):
            tb = t0 + ci * CHT
            pltpu.sync_copy(gated_hbm.at[pl.ds(tb, CHT)], bufc)
            return acc_chunk(bufc, tb, jnp.minimum(t1 - tb, CHT), s, e,
                             accs2)

        accs = lax.fori_loop(1, nch, cb, accs)
        denom = jnp.maximum(nrows, 1).astype(jnp.float32)
        a8 = list(accs)
        for shift in (8, 4, 2, 1):
            for r in range(8):
                tmp[pl.ds(r * 16, 16)] = a8[r]
            for r in range(8):
                a8[r] = a8[r] + tmp[pl.ds(r * 16 + shift, 16)]
        resv = zero16
        for r in range(8):
            resv = jnp.where(lane == r, a8[r][0], resv)
        res[pl.ds(g * 16, 16)] = resv / denom

    def quad_body(p, carry):
        g0 = 4 * p
        run_seg(g0, buf0, sem0, buf2, sem2, g0 + 2 < SEG_PER)
        run_seg(g0 + 1, buf1, sem1, buf3, sem3, g0 + 3 < SEG_PER)
        run_seg(g0 + 2, buf2, sem2, buf0, sem0, g0 + 4 < SEG_PER)
        run_seg(g0 + 3, buf3, sem3, buf1, sem1, g0 + 5 < SEG_PER)
        return carry

    lax.fori_loop(0, SEG_PER // 4, quad_body, 0)
    pltpu.sync_copy(res.at[pl.ds(0, SEG_PER * 16)],
                    out_hbm.at[pl.ds(t * SEG_PER * 16, SEG_PER * 16)])


def kernel(flat_features, row_splits, W_out, b_out, W_gate, b_gate, W_trans, b_trans):
    w_outT = W_out.astype(jnp.float32).T
    wpack = jnp.zeros((16, D), jnp.float32)
    wpack = wpack.at[0:8, :].set(W_trans.astype(jnp.float32).T)
    wpack = wpack.at[8, :].set(W_gate.astype(jnp.float32)[:, 0])
    b_out_col = b_out.astype(jnp.float32).reshape(D, 1)
    bias_pack = jnp.zeros((8, D), jnp.float32)
    bias_pack = bias_pack.at[:, 0].set(b_trans.astype(jnp.float32))
    bias_pack = bias_pack.at[0, 1].set(b_gate.astype(jnp.float32)[0])

    gated = _gated_tiles(flat_features, w_outT, wpack, b_out_col, bias_pack)

    splits_p = jnp.concatenate(
        [row_splits.astype(jnp.int32), jnp.full((7,), N, jnp.int32)])

    out_flat = _make_segmean()(gated, splits_p)
    return out_flat.reshape(G, 16)[:, :OUT]


# speedup vs baseline: 1.2104x; 1.0130x over previous
import functools

import jax
import jax.numpy as jnp
from jax import lax
from jax.experimental import pallas as pl
from jax.experimental.pallas import tpu as pltpu
from jax.experimental.pallas import tpu_sc as plsc

N = 320000
D = 128
G = 1024
OUT = 8

BLK = 32000
NBLK = N // BLK
N_PAD = N + BLK
TILES = N_PAD // 128

NW = 32
SEG_PER = G // NW
CHT = 4


def _mlp_body(x_ref, woutT_ref, wpack_ref, boutc_ref, bias_ref, out_ref):
    x = x_ref[...]
    wp = wpack_ref[...]
    wtransT = wp[0:8, :]
    wcombT = jnp.dot(wtransT, woutT_ref[...],
                     preferred_element_type=jnp.float32)
    wcatT = jnp.concatenate([wcombT, wp[8:16, :]], axis=0)
    yT = lax.dot_general(wcatT, x, (((1,), (1,)), ((), ())),
                         preferred_element_type=jnp.float32)
    b = bias_ref[...]
    bcT = jnp.dot(wtransT, boutc_ref[...],
                  preferred_element_type=jnp.float32) + b[:, 0:1]
    bg = b[0:1, 1:2]
    gate = jax.nn.sigmoid(yT[8:9, :] + bg)
    gatedT = gate * (yT[0:8, :] + bcT)
    for t in range(BLK // 128):
        out_ref[t] = gatedT[:, 128 * t:128 * (t + 1)]


def _gated_tiles(x, w_outT, wpack, b_out_col, bias_pack):
    return pl.pallas_call(
        _mlp_body,
        grid=(NBLK + 1,),
        in_specs=[
            pl.BlockSpec((BLK, D), lambda i: (jnp.minimum(i, NBLK - 1), 0)),
            pl.BlockSpec((D, D), lambda i: (0, 0)),
            pl.BlockSpec((16, D), lambda i: (0, 0)),
            pl.BlockSpec((D, 1), lambda i: (0, 0)),
            pl.BlockSpec((8, D), lambda i: (0, 0)),
        ],
        out_specs=pl.BlockSpec((BLK // 128, 8, 128), lambda i: (i, 0, 0)),
        out_shape=jax.ShapeDtypeStruct((TILES, 8, 128), jnp.float32),
    )(x, w_outT, wpack, b_out_col, bias_pack)


@functools.cache
def _make_segmean():
    return functools.partial(
        pl.kernel,
        out_type=jax.ShapeDtypeStruct((G * 16,), jnp.float32),
        mesh=plsc.VectorSubcoreMesh(core_axis_name="c", subcore_axis_name="s"),
        scratch_types=[
            pltpu.VMEM((48,), jnp.int32),
            pltpu.VMEM((CHT, 8, 128), jnp.float32),
            pltpu.VMEM((CHT, 8, 128), jnp.float32),
            pltpu.VMEM((CHT, 8, 128), jnp.float32),
            pltpu.VMEM((CHT, 8, 128), jnp.float32),
            pltpu.VMEM((CHT, 8, 128), jnp.float32),
            pltpu.SemaphoreType.DMA,
            pltpu.SemaphoreType.DMA,
            pltpu.SemaphoreType.DMA,
            pltpu.SemaphoreType.DMA,
            pltpu.VMEM((144,), jnp.float32),
            pltpu.VMEM((SEG_PER * 16,), jnp.float32),
        ],
    )(_segmean_body)


def _segmean_body(gated_hbm, splits_hbm, out_hbm, splits_v,
                  buf0, buf1, buf2, buf3, bufc,
                  sem0, sem1, sem2, sem3, tmp, res):
    cid = lax.axis_index("c")
    sid = lax.axis_index("s")
    t = sid * 2 + cid
    lane = lax.iota(jnp.int32, 16)
    zero16 = jnp.zeros((16,), jnp.float32)
    pltpu.sync_copy(splits_hbm.at[pl.ds(t * SEG_PER, 40)],
                    splits_v.at[pl.ds(0, 40)])

    def acc_chunk(buf, tbase, ntiles, s, e, accs0):
        def tile_body(tt, accs2):
            colbase = (tbase + tt) * 128
            cols = [jnp.full((16,), colbase + 16 * j, jnp.int32) + lane
                    for j in range(8)]
            masks = [(c >= s) & (c < e) for c in cols]
            out = []
            for r in range(8):
                a = accs2[r]
                for j in range(8):
                    v = buf[tt, r, pl.ds(16 * j, 16)]
                    a = a + jnp.where(masks[j], v, zero16)
                out.append(a)
            return tuple(out)

        return lax.fori_loop(0, ntiles, tile_body, accs0)

    sv0 = splits_v[pl.ds(0, 16)]
    pltpu.async_copy(
        gated_hbm.at[pl.ds(lax.shift_right_logical(sv0[0], 7), CHT)],
        buf0, sem0)
    pltpu.async_copy(
        gated_hbm.at[pl.ds(lax.shift_right_logical(sv0[1], 7), CHT)],
        buf1, sem1)

    def run_seg(g, buf_a, sem_a, buf_b, sem_b, prefetch):
        sv = splits_v[pl.ds(g, 16)]
        s = sv[0]
        e = sv[1]
        nrows = e - s
        t0 = lax.shift_right_logical(s, 7)
        t1 = lax.shift_right_logical(e + 127, 7)
        nch = lax.shift_right_logical(t1 - t0 + (CHT - 1), 2)

        @pl.when(prefetch)
        def _prefetch():
            pltpu.async_copy(
                gated_hbm.at[pl.ds(lax.shift_right_logical(sv[2], 7), CHT)],
                buf_b, sem_b)

        pltpu.make_async_copy(gated_hbm.at[pl.ds(0, CHT)],
                              buf_a, sem_a).wait()
        accs = acc_chunk(buf_a, t0, jnp.minimum(t1 - t0, CHT), s, e,
                         (zero16,) * 8)

        def cb(ci, accs2):
            tb = t0 + ci * CHT
            pltpu.sync_copy(gated_hbm.at[pl.ds(tb, CHT)], bufc)
            return acc_chunk(bufc, tb, jnp.minimum(t1 - tb, CHT), s, e,
                             accs2)

        accs = lax.fori_loop(1, nch, cb, accs)
        denom = jnp.maximum(nrows, 1).astype(jnp.float32)
        a8 = list(accs)
        for shift in (8, 4, 2, 1):
            for r in range(8):
                tmp[pl.ds(r * 16, 16)] = a8[r]
            for r in range(8):
                a8[r] = a8[r] + tmp[pl.ds(r * 16 + shift, 16)]
        resv = zero16
        for r in range(8):
            resv = jnp.where(lane == r, a8[r][0], resv)
        res[pl.ds(g * 16, 16)] = resv / denom

    def quad_body(p, carry):
        g0 = 4 * p
        run_seg(g0, buf0, sem0, buf2, sem2, g0 + 2 < SEG_PER)
        run_seg(g0 + 1, buf1, sem1, buf3, sem3, g0 + 3 < SEG_PER)
        run_seg(g0 + 2, buf2, sem2, buf0, sem0, g0 + 4 < SEG_PER)
        run_seg(g0 + 3, buf3, sem3, buf1, sem1, g0 + 5 < SEG_PER)
        return carry

    lax.fori_loop(0, SEG_PER // 4, quad_body, 0)
    pltpu.sync_copy(res.at[pl.ds(0, SEG_PER * 16)],
                    out_hbm.at[pl.ds(t * SEG_PER * 16, SEG_PER * 16)])


def kernel(flat_features, row_splits, W_out, b_out, W_gate, b_gate, W_trans, b_trans):
    w_outT = W_out.astype(jnp.float32).T
    wpack = jnp.zeros((16, D), jnp.float32)
    wpack = wpack.at[0:8, :].set(W_trans.astype(jnp.float32).T)
    wpack = wpack.at[8, :].set(W_gate.astype(jnp.float32)[:, 0])
    b_out_col = b_out.astype(jnp.float32).reshape(D, 1)
    bias_pack = jnp.zeros((8, D), jnp.float32)
    bias_pack = bias_pack.at[:, 0].set(b_trans.astype(jnp.float32))
    bias_pack = bias_pack.at[0, 1].set(b_gate.astype(jnp.float32)[0])

    gated = _gated_tiles(flat_features, w_outT, wpack, b_out_col, bias_pack)

    splits_p = jnp.concatenate(
        [row_splits.astype(jnp.int32), jnp.full((7,), N, jnp.int32)])

    out_flat = _make_segmean()(gated, splits_p)
    return out_flat.reshape(G, 16)[:, :OUT]


# speedup vs baseline: 1.2354x; 1.0207x over previous
import functools

import jax
import jax.numpy as jnp
from jax import lax
from jax.experimental import pallas as pl
from jax.experimental.pallas import tpu as pltpu
from jax.experimental.pallas import tpu_sc as plsc

N = 320000
D = 128
G = 1024
OUT = 8

BLK = 32000
NBLK = N // BLK
N_PAD = N + BLK
TILES = N_PAD // 128

NW = 32
SEG_PER = G // NW
CHT = 8


def _mlp_body(x_ref, woutT_ref, wpack_ref, boutc_ref, bias_ref, out_ref):
    x = x_ref[...]
    wp = wpack_ref[...]
    wtransT = wp[0:8, :]
    wcombT = jnp.dot(wtransT, woutT_ref[...],
                     preferred_element_type=jnp.float32)
    wcatT = jnp.concatenate([wcombT, wp[8:16, :]], axis=0)
    yT = lax.dot_general(wcatT, x, (((1,), (1,)), ((), ())),
                         preferred_element_type=jnp.float32)
    b = bias_ref[...]
    bcT = jnp.dot(wtransT, boutc_ref[...],
                  preferred_element_type=jnp.float32) + b[:, 0:1]
    bg = b[0:1, 1:2]
    gate = jax.nn.sigmoid(yT[8:9, :] + bg)
    gatedT = gate * (yT[0:8, :] + bcT)
    for t in range(BLK // 128):
        out_ref[t] = gatedT[:, 128 * t:128 * (t + 1)]


def _gated_tiles(x, w_outT, wpack, b_out_col, bias_pack):
    return pl.pallas_call(
        _mlp_body,
        grid=(NBLK + 1,),
        in_specs=[
            pl.BlockSpec((BLK, D), lambda i: (jnp.minimum(i, NBLK - 1), 0)),
            pl.BlockSpec((D, D), lambda i: (0, 0)),
            pl.BlockSpec((16, D), lambda i: (0, 0)),
            pl.BlockSpec((D, 1), lambda i: (0, 0)),
            pl.BlockSpec((8, D), lambda i: (0, 0)),
        ],
        out_specs=pl.BlockSpec((BLK // 128, 8, 128), lambda i: (i, 0, 0)),
        out_shape=jax.ShapeDtypeStruct((TILES, 8, 128), jnp.float32),
    )(x, w_outT, wpack, b_out_col, bias_pack)


@functools.cache
def _make_segmean():
    return functools.partial(
        pl.kernel,
        out_type=jax.ShapeDtypeStruct((G * 16,), jnp.float32),
        mesh=plsc.VectorSubcoreMesh(core_axis_name="c", subcore_axis_name="s"),
        scratch_types=[
            pltpu.VMEM((48,), jnp.int32),
            pltpu.VMEM((CHT, 8, 128), jnp.float32),
            pltpu.VMEM((CHT, 8, 128), jnp.float32),
            pltpu.VMEM((CHT, 8, 128), jnp.float32),
            pltpu.VMEM((CHT, 8, 128), jnp.float32),
            pltpu.VMEM((CHT, 8, 128), jnp.float32),
            pltpu.SemaphoreType.DMA,
            pltpu.SemaphoreType.DMA,
            pltpu.SemaphoreType.DMA,
            pltpu.SemaphoreType.DMA,
            pltpu.VMEM((144,), jnp.float32),
            pltpu.VMEM((SEG_PER * 16,), jnp.float32),
        ],
    )(_segmean_body)


def _segmean_body(gated_hbm, splits_hbm, out_hbm, splits_v,
                  buf0, buf1, buf2, buf3, bufc,
                  sem0, sem1, sem2, sem3, tmp, res):
    cid = lax.axis_index("c")
    sid = lax.axis_index("s")
    t = sid * 2 + cid
    lane = lax.iota(jnp.int32, 16)
    zero16 = jnp.zeros((16,), jnp.float32)
    pltpu.sync_copy(splits_hbm.at[pl.ds(t * SEG_PER, 40)],
                    splits_v.at[pl.ds(0, 40)])

    def acc_chunk(buf, tbase, ntiles, s, e, accs0):
        def tile_body(tt, accs2):
            colbase = (tbase + tt) * 128
            cols = [jnp.full((16,), colbase + 16 * j, jnp.int32) + lane
                    for j in range(8)]
            masks = [(c >= s) & (c < e) for c in cols]
            out = []
            for r in range(8):
                a = accs2[r]
                for j in range(8):
                    v = buf[tt, r, pl.ds(16 * j, 16)]
                    a = a + jnp.where(masks[j], v, zero16)
                out.append(a)
            return tuple(out)

        return lax.fori_loop(0, ntiles, tile_body, accs0)

    sv0 = splits_v[pl.ds(0, 16)]
    pltpu.async_copy(
        gated_hbm.at[pl.ds(lax.shift_right_logical(sv0[0], 7), CHT)],
        buf0, sem0)
    pltpu.async_copy(
        gated_hbm.at[pl.ds(lax.shift_right_logical(sv0[1], 7), CHT)],
        buf1, sem1)

    def run_seg(g, buf_a, sem_a, buf_b, sem_b, prefetch):
        sv = splits_v[pl.ds(g, 16)]
        s = sv[0]
        e = sv[1]
        nrows = e - s
        t0 = lax.shift_right_logical(s, 7)
        t1 = lax.shift_right_logical(e + 127, 7)
        nch = lax.shift_right_logical(t1 - t0 + (CHT - 1), 3)

        @pl.when(prefetch)
        def _prefetch():
            pltpu.async_copy(
                gated_hbm.at[pl.ds(lax.shift_right_logical(sv[2], 7), CHT)],
                buf_b, sem_b)

        pltpu.make_async_copy(gated_hbm.at[pl.ds(0, CHT)],
                              buf_a, sem_a).wait()
        accs = acc_chunk(buf_a, t0, jnp.minimum(t1 - t0, CHT), s, e,
                         (zero16,) * 8)

        def cb(ci, accs2):
            tb = t0 + ci * CHT
            pltpu.sync_copy(gated_hbm.at[pl.ds(tb, CHT)], bufc)
            return acc_chunk(bufc, tb, jnp.minimum(t1 - tb, CHT), s, e,
                             accs2)

        accs = lax.fori_loop(1, nch, cb, accs)
        denom = jnp.maximum(nrows, 1).astype(jnp.float32)
        a8 = list(accs)
        for shift in (8, 4, 2, 1):
            for r in range(8):
                tmp[pl.ds(r * 16, 16)] = a8[r]
            for r in range(8):
                a8[r] = a8[r] + tmp[pl.ds(r * 16 + shift, 16)]
        resv = zero16
        for r in range(8):
            resv = jnp.where(lane == r, a8[r][0], resv)
        res[pl.ds(g * 16, 16)] = resv / denom

    def quad_body(p, carry):
        g0 = 4 * p
        run_seg(g0, buf0, sem0, buf2, sem2, g0 + 2 < SEG_PER)
        run_seg(g0 + 1, buf1, sem1, buf3, sem3, g0 + 3 < SEG_PER)
        run_seg(g0 + 2, buf2, sem2, buf0, sem0, g0 + 4 < SEG_PER)
        run_seg(g0 + 3, buf3, sem3, buf1, sem1, g0 + 5 < SEG_PER)
        return carry

    lax.fori_loop(0, SEG_PER // 4, quad_body, 0)
    pltpu.sync_copy(res.at[pl.ds(0, SEG_PER * 16)],
                    out_hbm.at[pl.ds(t * SEG_PER * 16, SEG_PER * 16)])


def kernel(flat_features, row_splits, W_out, b_out, W_gate, b_gate, W_trans, b_trans):
    w_outT = W_out.astype(jnp.float32).T
    wpack = jnp.zeros((16, D), jnp.float32)
    wpack = wpack.at[0:8, :].set(W_trans.astype(jnp.float32).T)
    wpack = wpack.at[8, :].set(W_gate.astype(jnp.float32)[:, 0])
    b_out_col = b_out.astype(jnp.float32).reshape(D, 1)
    bias_pack = jnp.zeros((8, D), jnp.float32)
    bias_pack = bias_pack.at[:, 0].set(b_trans.astype(jnp.float32))
    bias_pack = bias_pack.at[0, 1].set(b_gate.astype(jnp.float32)[0])

    gated = _gated_tiles(flat_features, w_outT, wpack, b_out_col, bias_pack)

    splits_p = jnp.concatenate(
        [row_splits.astype(jnp.int32), jnp.full((7,), N, jnp.int32)])

    out_flat = _make_segmean()(gated, splits_p)
    return out_flat.reshape(G, 16)[:, :OUT]
